# Initial kernel scaffold; baseline (speedup 1.0000x reference)
import jax
import jax.numpy as jnp
from jax.experimental import pallas as pl


def kernel(x, adj, W, b):
    raise NotImplementedError("write your pallas kernel here")


# speedup vs baseline: 5.4907x; 5.4907x over previous
import functools

import jax
import jax.numpy as jnp
from jax import lax
from jax.experimental import pallas as pl
from jax.experimental.pallas import tpu as pltpu
from jax.experimental.pallas import tpu_sc as plsc

B, N, C = 8, 2048, 512
K = N // 2


_RANK_CH = 256


def _score_rank_body(x_ref, w_ref, b_ref, s_in_ref, score_ref, rank_ref):
    xb = x_ref[0]
    w = w_ref[0]
    score_ref[0, 0, :] = jnp.sum(xb * w[None, :], axis=1) + b_ref[0]
    s = s_in_ref[0, 0, :]
    iota_n = lax.iota(jnp.int32, N)
    for c in range(N // _RANK_CH):
        sc = s[c * _RANK_CH:(c + 1) * _RANK_CH]
        ic = iota_n[c * _RANK_CH:(c + 1) * _RANK_CH]
        gt = (s[None, :] > sc[:, None])
        eq = (s[None, :] == sc[:, None]) & (iota_n[None, :] < ic[:, None])
        cnt = jnp.sum(gt.astype(jnp.int32) + eq.astype(jnp.int32), axis=1)
        rank_ref[0, 0, c * _RANK_CH:(c + 1) * _RANK_CH] = cnt


def _score_rank(x, W, b, s_in):
    return pl.pallas_call(
        _score_rank_body,
        grid=(B,),
        in_specs=[
            pl.BlockSpec((1, N, C), lambda i: (i, 0, 0)),
            pl.BlockSpec((1, C), lambda i: (0, 0)),
            pl.BlockSpec((1,), lambda i: (0,)),
            pl.BlockSpec((1, 1, N), lambda i: (i, 0, 0)),
        ],
        out_specs=[
            pl.BlockSpec((1, 1, N), lambda i: (i, 0, 0)),
            pl.BlockSpec((1, 1, N), lambda i: (i, 0, 0)),
        ],
        out_shape=[
            jax.ShapeDtypeStruct((B, 1, N), jnp.float32),
            jax.ShapeDtypeStruct((B, 1, N), jnp.int32),
        ],
    )(x, W, b, s_in)


_NW = 32
_TPB = _NW // B
_SLOTS = K // _TPB
_XCH = 32
_ACH = 8


def _sc_body(rank_hbm, x_hbm, adj_hbm, newx_hbm, newadj_hbm,
             rank_v, sidx_v, idx_v, xbuf, abuf, obuf, sem):
    nc = 2
    wid = lax.axis_index("s") * nc + lax.axis_index("c")
    b = wid // _TPB
    q = wid % _TPB

    pltpu.sync_copy(rank_hbm.at[b], rank_v)
    iota = lax.iota(jnp.int32, 16)

    def build(i, _):
        rv = rank_v[pl.ds(i * 16, 16)]
        m = rv < K
        rvc = jnp.where(m, rv, 0)
        plsc.store_scatter(sidx_v, [rvc], iota + i * 16, mask=m)
        return 0

    lax.fori_loop(0, N // 16, build, 0, unroll=4)

    def mkidx(i, _):
        sv = sidx_v[pl.ds(q * _SLOTS + i * 16, 16)]
        idx_v[pl.ds(i * 16, 16)] = sv + b * N
        return 0

    lax.fori_loop(0, _SLOTS // 16, mkidx, 0, unroll=4)

    out_base = b * K + q * _SLOTS

    def xchunk(ci, _):
        cp = pltpu.async_copy(x_hbm.at[idx_v.at[pl.ds(ci * _XCH, _XCH)]],
                              xbuf, sem)
        cp.wait()
        pltpu.sync_copy(xbuf, newx_hbm.at[pl.ds(out_base + ci * _XCH, _XCH)])
        return 0

    lax.fori_loop(0, _SLOTS // _XCH, xchunk, 0)

    def achunk(ci, _):
        cp = pltpu.async_copy(adj_hbm.at[idx_v.at[pl.ds(ci * _ACH, _ACH)]],
                              abuf, sem)
        cp.wait()

        def row(j, _):
            jv = jnp.full((16,), j, jnp.int32)

            def col(cc, _):
                cidx = sidx_v[pl.ds(cc * 16, 16)]
                vals = plsc.load_gather(abuf, [jv, cidx])
                plsc.store_scatter(obuf, [jv, iota + cc * 16], vals)
                return 0

            lax.fori_loop(0, K // 16, col, 0, unroll=8)
            return 0

        lax.fori_loop(0, _ACH, row, 0)
        pltpu.sync_copy(obuf, newadj_hbm.at[pl.ds(out_base + ci * _ACH, _ACH)])
        return 0

    lax.fori_loop(0, _SLOTS // _ACH, achunk, 0)


@functools.partial(jax.jit, static_argnames=())
def _sc_gather(rank, x2d, adj2d):
    mesh = plsc.VectorSubcoreMesh(core_axis_name="c", subcore_axis_name="s")
    return pl.kernel(
        _sc_body,
        out_type=[
            jax.ShapeDtypeStruct((B * K, C), jnp.float32),
            jax.ShapeDtypeStruct((B * K, K), jnp.float32),
        ],
        mesh=mesh,
        scratch_types=[
            pltpu.VMEM((N,), jnp.int32),
            pltpu.VMEM((K,), jnp.int32),
            pltpu.VMEM((_SLOTS,), jnp.int32),
            pltpu.VMEM((_XCH, C), jnp.float32),
            pltpu.VMEM((_ACH, N), jnp.float32),
            pltpu.VMEM((_ACH, K), jnp.float32),
            pltpu.SemaphoreType.DMA,
        ],
        compiler_params=pltpu.CompilerParams(needs_layout_passes=False),
    )(rank, x2d, adj2d)


def kernel(x, adj, W, b):
    s_parity = (x @ W.T + b)[..., 0]
    score3, rank3 = _score_rank(x, W, b, s_parity.reshape(B, 1, N))
    score = score3.reshape(B, N)
    rank = rank3.reshape(B, N)
    new_x, new_adj = _sc_gather(rank,
                                x.reshape(B * N, C),
                                adj.reshape(B * N, N))
    return (new_x.reshape(B, K, C), new_adj.reshape(B, K, K), score)


# speedup vs baseline: 13.3637x; 2.4339x over previous
import functools

import jax
import jax.numpy as jnp
from jax import lax
from jax.experimental import pallas as pl
from jax.experimental.pallas import tpu as pltpu
from jax.experimental.pallas import tpu_sc as plsc

B, N, C = 8, 2048, 512
K = N // 2


_RANK_CH = 256


def _score_rank_body(x_ref, w_ref, b_ref, s_in_ref, score_ref, rank_ref):
    xb = x_ref[0]
    w = w_ref[0]
    score_ref[0, 0, :] = jnp.sum(xb * w[None, :], axis=1) + b_ref[0]
    s = s_in_ref[0, 0, :]
    iota_n = lax.iota(jnp.int32, N)
    for c in range(N // _RANK_CH):
        sc = s[c * _RANK_CH:(c + 1) * _RANK_CH]
        ic = iota_n[c * _RANK_CH:(c + 1) * _RANK_CH]
        gt = (s[None, :] > sc[:, None])
        eq = (s[None, :] == sc[:, None]) & (iota_n[None, :] < ic[:, None])
        cnt = jnp.sum(gt.astype(jnp.int32) + eq.astype(jnp.int32), axis=1)
        rank_ref[0, 0, c * _RANK_CH:(c + 1) * _RANK_CH] = cnt


def _score_rank(x, W, b, s_in):
    return pl.pallas_call(
        _score_rank_body,
        grid=(B,),
        in_specs=[
            pl.BlockSpec((1, N, C), lambda i: (i, 0, 0)),
            pl.BlockSpec((1, C), lambda i: (0, 0)),
            pl.BlockSpec((1,), lambda i: (0,)),
            pl.BlockSpec((1, 1, N), lambda i: (i, 0, 0)),
        ],
        out_specs=[
            pl.BlockSpec((1, 1, N), lambda i: (i, 0, 0)),
            pl.BlockSpec((1, 1, N), lambda i: (i, 0, 0)),
        ],
        out_shape=[
            jax.ShapeDtypeStruct((B, 1, N), jnp.float32),
            jax.ShapeDtypeStruct((B, 1, N), jnp.int32),
        ],
    )(x, W, b, s_in)


_NW = 32
_TPB = _NW // B
_SLOTS = K // _TPB
_XCH = 32
_ACH = 8


def _sc_body(rank_hbm, x_hbm, adj_hbm, newx_hbm, newadj_hbm,
             rank_v, sidx_v, idx_v, xbuf0, xbuf1, abuf0, abuf1, obuf,
             semx0, semx1, sema0, sema1):
    nc = 2
    wid = lax.axis_index("s") * nc + lax.axis_index("c")
    b = wid // _TPB
    q = wid % _TPB

    pltpu.sync_copy(rank_hbm.at[b], rank_v)
    iota = lax.iota(jnp.int32, 16)

    @plsc.parallel_loop(0, N // 16, unroll=4)
    def _build(i):
        rv = rank_v[pl.ds(i * 16, 16)]
        m = rv < K
        rvc = jnp.where(m, rv, 0)
        plsc.store_scatter(sidx_v, [rvc], iota + i * 16, mask=m)

    @plsc.parallel_loop(0, _SLOTS // 16, unroll=4)
    def _mkidx(i):
        sv = sidx_v[pl.ds(q * _SLOTS + i * 16, 16)]
        idx_v[pl.ds(i * 16, 16)] = sv + b * N

    out_base = b * K + q * _SLOTS

    xbufs, xsems = (xbuf0, xbuf1), (semx0, semx1)
    nxc = _SLOTS // _XCH

    def xin(ci, par):
        return pltpu.async_copy(
            x_hbm.at[idx_v.at[pl.ds(ci * _XCH, _XCH)]], xbufs[par], xsems[par])

    pend = [xin(0, 0), xin(1, 1)]
    for ci in range(nxc):
        par = ci % 2
        pend[par].wait()
        pltpu.sync_copy(xbufs[par],
                        newx_hbm.at[pl.ds(out_base + ci * _XCH, _XCH)])
        if ci + 2 < nxc:
            pend[par] = xin(ci + 2, par)

    abufs, asems = (abuf0, abuf1), (sema0, sema1)
    nac = _SLOTS // _ACH

    def ain_src(ci):
        return adj_hbm.at[idx_v.at[pl.ds(ci * _ACH, _ACH)]]

    def colgather(buf):
        @plsc.parallel_loop(0, K // 16, unroll=2)
        def _cols(cc):
            cidx = sidx_v[pl.ds(cc * 16, 16)]
            for j in range(_ACH):
                jv = jnp.full((16,), j, jnp.int32)
                vals = plsc.load_gather(buf, [jv, cidx])
                obuf[j, pl.ds(cc * 16, 16)] = vals

    pltpu.async_copy(ain_src(0), abuf0, sema0)
    pltpu.async_copy(ain_src(1), abuf1, sema1)

    def apair(p, _):
        for par in range(2):
            ci = 2 * p + par
            pltpu.make_async_copy(ain_src(ci), abufs[par], asems[par]).wait()
            colgather(abufs[par])
            pltpu.sync_copy(obuf,
                            newadj_hbm.at[pl.ds(out_base + ci * _ACH, _ACH)])
            pltpu.async_copy(ain_src(ci + 2), abufs[par], asems[par])
        return 0

    lax.fori_loop(0, nac // 2 - 1, apair, 0)
    for ci in (nac - 2, nac - 1):
        par = ci % 2
        pltpu.make_async_copy(ain_src(ci), abufs[par], asems[par]).wait()
        colgather(abufs[par])
        pltpu.sync_copy(obuf,
                        newadj_hbm.at[pl.ds(out_base + ci * _ACH, _ACH)])


@functools.partial(jax.jit, static_argnames=())
def _sc_gather(rank, x2d, adj2d):
    mesh = plsc.VectorSubcoreMesh(core_axis_name="c", subcore_axis_name="s")
    return pl.kernel(
        _sc_body,
        out_type=[
            jax.ShapeDtypeStruct((B * K, C), jnp.float32),
            jax.ShapeDtypeStruct((B * K, K), jnp.float32),
        ],
        mesh=mesh,
        scratch_types=[
            pltpu.VMEM((N,), jnp.int32),
            pltpu.VMEM((K,), jnp.int32),
            pltpu.VMEM((_SLOTS,), jnp.int32),
            pltpu.VMEM((_XCH, C), jnp.float32),
            pltpu.VMEM((_XCH, C), jnp.float32),
            pltpu.VMEM((_ACH, N), jnp.float32),
            pltpu.VMEM((_ACH, N), jnp.float32),
            pltpu.VMEM((_ACH, K), jnp.float32),
            pltpu.SemaphoreType.DMA,
            pltpu.SemaphoreType.DMA,
            pltpu.SemaphoreType.DMA,
            pltpu.SemaphoreType.DMA,
        ],
        compiler_params=pltpu.CompilerParams(needs_layout_passes=False),
    )(rank, x2d, adj2d)


def kernel(x, adj, W, b):
    s_parity = (x @ W.T + b)[..., 0]
    score3, rank3 = _score_rank(x, W, b, s_parity.reshape(B, 1, N))
    score = score3.reshape(B, N)
    rank = rank3.reshape(B, N)
    new_x, new_adj = _sc_gather(rank,
                                x.reshape(B * N, C),
                                adj.reshape(B * N, N))
    return (new_x.reshape(B, K, C), new_adj.reshape(B, K, K), score)


# speedup vs baseline: 14.6220x; 1.0942x over previous
import functools

import jax
import jax.numpy as jnp
from jax import lax
from jax.experimental import pallas as pl
from jax.experimental.pallas import tpu as pltpu
from jax.experimental.pallas import tpu_sc as plsc

B, N, C = 8, 2048, 512
K = N // 2


_RANK_CH = 256


def _score_rank_body(x_ref, w_ref, b_ref, s_in_ref, score_ref, rank_ref):
    xb = x_ref[0]
    w = w_ref[0]
    score_ref[0, 0, :] = jnp.sum(xb * w[None, :], axis=1) + b_ref[0]
    s = s_in_ref[0, 0, :]
    iota_n = lax.iota(jnp.int32, N)
    ones = jnp.ones((N, 1), jnp.float32)
    for c in range(N // _RANK_CH):
        sc = s[c * _RANK_CH:(c + 1) * _RANK_CH]
        ic = iota_n[c * _RANK_CH:(c + 1) * _RANK_CH]
        ge = (s[None, :] >= sc[:, None])
        gt = (s[None, :] > sc[:, None])
        jl = (iota_n[None, :] < ic[:, None])
        cmp = jnp.where(jl, ge.astype(jnp.float32),
                        gt.astype(jnp.float32))
        cnt = jax.lax.dot(cmp, ones,
                          preferred_element_type=jnp.float32)[:, 0]
        rank_ref[0, 0, c * _RANK_CH:(c + 1) * _RANK_CH] = cnt.astype(jnp.int32)


def _score_rank(x, W, b, s_in):
    return pl.pallas_call(
        _score_rank_body,
        grid=(B,),
        in_specs=[
            pl.BlockSpec((1, N, C), lambda i: (i, 0, 0)),
            pl.BlockSpec((1, C), lambda i: (0, 0)),
            pl.BlockSpec((1,), lambda i: (0,)),
            pl.BlockSpec((1, 1, N), lambda i: (i, 0, 0)),
        ],
        out_specs=[
            pl.BlockSpec((1, 1, N), lambda i: (i, 0, 0)),
            pl.BlockSpec((1, 1, N), lambda i: (i, 0, 0)),
        ],
        out_shape=[
            jax.ShapeDtypeStruct((B, 1, N), jnp.float32),
            jax.ShapeDtypeStruct((B, 1, N), jnp.int32),
        ],
    )(x, W, b, s_in)


_NW = 32
_TPB = _NW // B
_SLOTS = K // _TPB
_XCH = 32
_ACH = 16


def _sc_body(rank_hbm, x_hbm, adj_hbm, newx_hbm, newadj_hbm,
             rank_v, sidx_v, idx_v, xbuf0, xbuf1, abuf0, abuf1, obuf,
             semx0, semx1, sema0, sema1):
    nc = 2
    wid = lax.axis_index("s") * nc + lax.axis_index("c")
    b = wid // _TPB
    q = wid % _TPB

    pltpu.sync_copy(rank_hbm.at[b], rank_v)
    iota = lax.iota(jnp.int32, 16)

    @plsc.parallel_loop(0, N // 16, unroll=4)
    def _build(i):
        rv = rank_v[pl.ds(i * 16, 16)]
        m = rv < K
        rvc = jnp.where(m, rv, 0)
        plsc.store_scatter(sidx_v, [rvc], iota + i * 16, mask=m)

    @plsc.parallel_loop(0, _SLOTS // 16, unroll=4)
    def _mkidx(i):
        sv = sidx_v[pl.ds(q * _SLOTS + i * 16, 16)]
        idx_v[pl.ds(i * 16, 16)] = sv + b * N

    out_base = b * K + q * _SLOTS

    xbufs, xsems = (xbuf0, xbuf1), (semx0, semx1)
    nxc = _SLOTS // _XCH

    def xin(ci, par):
        return pltpu.async_copy(
            x_hbm.at[idx_v.at[pl.ds(ci * _XCH, _XCH)]], xbufs[par], xsems[par])

    pend = [xin(0, 0), xin(1, 1)]
    for ci in range(nxc):
        par = ci % 2
        pend[par].wait()
        pltpu.sync_copy(xbufs[par],
                        newx_hbm.at[pl.ds(out_base + ci * _XCH, _XCH)])
        if ci + 2 < nxc:
            pend[par] = xin(ci + 2, par)

    abufs, asems = (abuf0, abuf1), (sema0, sema1)
    nac = _SLOTS // _ACH

    def ain_src(ci):
        return adj_hbm.at[idx_v.at[pl.ds(ci * _ACH, _ACH)]]

    def colgather(buf):
        @plsc.parallel_loop(0, K // 16, unroll=2)
        def _cols(cc):
            cidx = sidx_v[pl.ds(cc * 16, 16)]
            for j in range(_ACH):
                jv = jnp.full((16,), j, jnp.int32)
                vals = plsc.load_gather(buf, [jv, cidx])
                obuf[j, pl.ds(cc * 16, 16)] = vals

    pltpu.async_copy(ain_src(0), abuf0, sema0)
    pltpu.async_copy(ain_src(1), abuf1, sema1)

    def apair(p, _):
        for par in range(2):
            ci = 2 * p + par
            pltpu.make_async_copy(ain_src(ci), abufs[par], asems[par]).wait()
            colgather(abufs[par])
            pltpu.sync_copy(obuf,
                            newadj_hbm.at[pl.ds(out_base + ci * _ACH, _ACH)])
            pltpu.async_copy(ain_src(ci + 2), abufs[par], asems[par])
        return 0

    lax.fori_loop(0, nac // 2 - 1, apair, 0)
    for ci in (nac - 2, nac - 1):
        par = ci % 2
        pltpu.make_async_copy(ain_src(ci), abufs[par], asems[par]).wait()
        colgather(abufs[par])
        pltpu.sync_copy(obuf,
                        newadj_hbm.at[pl.ds(out_base + ci * _ACH, _ACH)])


@functools.partial(jax.jit, static_argnames=())
def _sc_gather(rank, x2d, adj2d):
    mesh = plsc.VectorSubcoreMesh(core_axis_name="c", subcore_axis_name="s")
    return pl.kernel(
        _sc_body,
        out_type=[
            jax.ShapeDtypeStruct((B * K, C), jnp.float32),
            jax.ShapeDtypeStruct((B * K, K), jnp.float32),
        ],
        mesh=mesh,
        scratch_types=[
            pltpu.VMEM((N,), jnp.int32),
            pltpu.VMEM((K,), jnp.int32),
            pltpu.VMEM((_SLOTS,), jnp.int32),
            pltpu.VMEM((_XCH, C), jnp.float32),
            pltpu.VMEM((_XCH, C), jnp.float32),
            pltpu.VMEM((_ACH, N), jnp.float32),
            pltpu.VMEM((_ACH, N), jnp.float32),
            pltpu.VMEM((_ACH, K), jnp.float32),
            pltpu.SemaphoreType.DMA,
            pltpu.SemaphoreType.DMA,
            pltpu.SemaphoreType.DMA,
            pltpu.SemaphoreType.DMA,
        ],
        compiler_params=pltpu.CompilerParams(needs_layout_passes=False),
    )(rank, x2d, adj2d)


def kernel(x, adj, W, b):
    s_parity = (x @ W.T + b)[..., 0]
    score3, rank3 = _score_rank(x, W, b, s_parity.reshape(B, 1, N))
    score = score3.reshape(B, N)
    rank = rank3.reshape(B, N)
    new_x, new_adj = _sc_gather(rank,
                                x.reshape(B * N, C),
                                adj.reshape(B * N, N))
    return (new_x.reshape(B, K, C), new_adj.reshape(B, K, K), score)
